# baseline (speedup 1.0000x reference)
import jax
import jax.numpy as jnp
from jax import lax
from jax.experimental import pallas as pl
from jax.experimental.pallas import tpu as pltpu
from jax.experimental.pallas import tpu_sc as plsc

VOCAB = 1000
VOCAB_PAD = 1024
BATCH = 1024
SEQ = 50

NC = 2
NS = 16
NW = NC * NS

BATCH_PER_W = BATCH // NW
SEQ_PAD = 64
GROWS = 56

HEAD = 896
TILE = 128
RHEAD = 48
RTILE = 8


def _gather_kernel(
    table_hbm, idx_hbm, out_hbm, idx_v, gbuf0, gbuf1, gsem0, gsem1, ssem0, ssem1
):
    cid = lax.axis_index("c")
    sid = lax.axis_index("s")
    wid = sid * NC + cid
    base_b = wid * BATCH_PER_W

    pltpu.sync_copy(idx_hbm.at[pl.ds(wid * BATCH_PER_W * SEQ_PAD,
                                     BATCH_PER_W * SEQ_PAD)], idx_v)

    tail_col = pl.multiple_of(HEAD + 0 * wid, TILE)
    tail_row = pl.multiple_of(RHEAD + 0 * wid, RTILE)

    def gather_start(gbuf, gsem, c):
        pltpu.make_async_copy(
            table_hbm.at[idx_v.at[pl.ds(c * SEQ_PAD, GROWS)]], gbuf, gsem
        ).start()

    def store_copies(gbuf, ssem, c):
        dst = out_hbm.at[base_b + c]
        return (
            pltpu.make_async_copy(
                gbuf.at[pl.ds(0, RHEAD), pl.ds(0, HEAD)],
                dst.at[pl.ds(0, RHEAD), pl.ds(0, HEAD)], ssem),
            pltpu.make_async_copy(
                gbuf.at[pl.ds(0, RHEAD), pl.ds(HEAD, TILE)],
                dst.at[pl.ds(0, RHEAD), pl.ds(tail_col, TILE)], ssem),
            pltpu.make_async_copy(
                gbuf.at[pl.ds(RHEAD, RTILE), pl.ds(0, HEAD)],
                dst.at[pl.ds(tail_row, RTILE), pl.ds(0, HEAD)], ssem),
            pltpu.make_async_copy(
                gbuf.at[pl.ds(RHEAD, RTILE), pl.ds(HEAD, TILE)],
                dst.at[pl.ds(tail_row, RTILE), pl.ds(tail_col, TILE)], ssem),
        )

    bufs = ((gbuf0, gsem0, ssem0), (gbuf1, gsem1, ssem1))

    for b in range(2):
        gbuf, gsem, _ = bufs[b]
        gather_start(gbuf, gsem, b)

    @pl.loop(0, BATCH_PER_W // 2)
    def _(p):
        for b in range(2):
            gbuf, gsem, ssem = bufs[b]
            c = p * 2 + b

            pltpu.make_async_copy(
                table_hbm.at[idx_v.at[pl.ds(c * SEQ_PAD, GROWS)]], gbuf, gsem
            ).wait()

            for cp in store_copies(gbuf, ssem, c):
                cp.start()

            @pl.when(c + 2 < BATCH_PER_W)
            def _():
                for cp in store_copies(gbuf, ssem, c):
                    cp.wait()
                gather_start(gbuf, gsem, c + 2)

    for b in range(2):
        gbuf, _, ssem = bufs[b]
        for cp in store_copies(gbuf, ssem, BATCH_PER_W - 2 + b):
            cp.wait()


@jax.jit
def _gather(table_pad, idx_pad_flat):
    mesh = plsc.VectorSubcoreMesh(core_axis_name="c", subcore_axis_name="s")
    k = pl.kernel(
        _gather_kernel,
        out_type=jax.ShapeDtypeStruct((BATCH, SEQ, VOCAB), jnp.float32),
        mesh=mesh,
        compiler_params=pltpu.CompilerParams(disable_bounds_checks=True),
        scratch_types=[
            pltpu.VMEM((BATCH_PER_W * SEQ_PAD,), jnp.int32),
            pltpu.VMEM((GROWS, VOCAB_PAD), jnp.float32),
            pltpu.VMEM((GROWS, VOCAB_PAD), jnp.float32),
            pltpu.SemaphoreType.DMA,
            pltpu.SemaphoreType.DMA,
            pltpu.SemaphoreType.DMA,
            pltpu.SemaphoreType.DMA,
        ],
    )
    return k(table_pad, idx_pad_flat)


def kernel(table, idx):
    table_pad = jnp.pad(table, ((0, 0), (0, VOCAB_PAD - VOCAB)))
    idx_pad = jnp.pad(idx, ((0, 0), (0, SEQ_PAD - SEQ)))
    return _gather(table_pad, idx_pad.reshape(-1))


# speedup vs baseline: 1.0119x; 1.0119x over previous
import jax
import jax.numpy as jnp
from jax import lax
from jax.experimental import pallas as pl
from jax.experimental.pallas import tpu as pltpu
from jax.experimental.pallas import tpu_sc as plsc

VOCAB = 1000
VOCAB_PAD = 1024
BATCH = 1024
SEQ = 50

NC = 2
NS = 16
NW = NC * NS

BATCH_PER_W = BATCH // NW
SEQ_PAD = 64
GROWS = 56

HEAD = 896
TILE = 128
RHEAD = 48
RTILE = 8


def _gather_kernel(
    table_hbm, idx_hbm, out_hbm, idx_v, gbuf0, gbuf1, gsem0, gsem1, ssem0, ssem1
):
    cid = lax.axis_index("c")
    sid = lax.axis_index("s")
    wid = sid * NC + cid
    base_b = wid * BATCH_PER_W

    pltpu.sync_copy(idx_hbm.at[pl.ds(wid * BATCH_PER_W * SEQ_PAD,
                                     BATCH_PER_W * SEQ_PAD)], idx_v)

    tail_col = pl.multiple_of(HEAD + 0 * wid, TILE)
    tail_row = pl.multiple_of(RHEAD + 0 * wid, RTILE)

    def gather_start(gbuf, gsem, c):
        pltpu.make_async_copy(
            table_hbm.at[idx_v.at[pl.ds(c * SEQ_PAD, GROWS)]], gbuf, gsem
        ).start()

    row0 = pl.multiple_of(0 * wid, RTILE)

    def store_copies(gbuf, ssem, c):
        dst = out_hbm.at[base_b + c]
        return (
            pltpu.make_async_copy(
                gbuf.at[:, pl.ds(0, HEAD)],
                dst.at[pl.ds(row0, GROWS), pl.ds(0, HEAD)], ssem),
            pltpu.make_async_copy(
                gbuf.at[:, pl.ds(HEAD, TILE)],
                dst.at[pl.ds(row0, GROWS), pl.ds(tail_col, TILE)], ssem),
        )

    bufs = ((gbuf0, gsem0, ssem0), (gbuf1, gsem1, ssem1))

    for b in range(2):
        gbuf, gsem, _ = bufs[b]
        gather_start(gbuf, gsem, b)

    @pl.loop(0, BATCH_PER_W // 2)
    def _(p):
        for b in range(2):
            gbuf, gsem, ssem = bufs[b]
            c = p * 2 + b

            pltpu.make_async_copy(
                table_hbm.at[idx_v.at[pl.ds(c * SEQ_PAD, GROWS)]], gbuf, gsem
            ).wait()

            for cp in store_copies(gbuf, ssem, c):
                cp.start()

            @pl.when(c + 2 < BATCH_PER_W)
            def _():
                for cp in store_copies(gbuf, ssem, c):
                    cp.wait()
                gather_start(gbuf, gsem, c + 2)

    for b in range(2):
        gbuf, _, ssem = bufs[b]
        for cp in store_copies(gbuf, ssem, BATCH_PER_W - 2 + b):
            cp.wait()


@jax.jit
def _gather(table_pad, idx_pad_flat):
    mesh = plsc.VectorSubcoreMesh(core_axis_name="c", subcore_axis_name="s")
    k = pl.kernel(
        _gather_kernel,
        out_type=jax.ShapeDtypeStruct((BATCH, SEQ, VOCAB), jnp.float32),
        mesh=mesh,
        compiler_params=pltpu.CompilerParams(disable_bounds_checks=True),
        scratch_types=[
            pltpu.VMEM((BATCH_PER_W * SEQ_PAD,), jnp.int32),
            pltpu.VMEM((GROWS, VOCAB_PAD), jnp.float32),
            pltpu.VMEM((GROWS, VOCAB_PAD), jnp.float32),
            pltpu.SemaphoreType.DMA,
            pltpu.SemaphoreType.DMA,
            pltpu.SemaphoreType.DMA,
            pltpu.SemaphoreType.DMA,
        ],
    )
    return k(table_pad, idx_pad_flat)


def kernel(table, idx):
    table_pad = jnp.pad(table, ((0, 0), (0, VOCAB_PAD - VOCAB)))
    idx_pad = jnp.pad(idx, ((0, 0), (0, SEQ_PAD - SEQ)))
    return _gather(table_pad, idx_pad.reshape(-1))


# speedup vs baseline: 1.0954x; 1.0826x over previous
import jax
import jax.numpy as jnp
from jax import lax
from jax.experimental import pallas as pl
from jax.experimental.pallas import tpu as pltpu
from jax.experimental.pallas import tpu_sc as plsc

VOCAB = 1000
VOCAB_PAD = 1024
BATCH = 1024
SEQ = 50

NC = 2
NS = 16
NW = NC * NS

BATCH_PER_W = BATCH // NW
SEQ_PAD = 64
GROWS = 56

HEAD = 896
TILE = 128
RHEAD = 48
RTILE = 8


def _gather_kernel(
    table_hbm, idx_hbm, out_hbm, idx_v, gbuf0, gbuf1, gsem0, gsem1, ssem0, ssem1
):
    cid = lax.axis_index("c")
    sid = lax.axis_index("s")
    wid = sid * NC + cid
    base_b = wid * BATCH_PER_W

    pltpu.sync_copy(idx_hbm.at[pl.ds(wid * BATCH_PER_W * SEQ_PAD,
                                     BATCH_PER_W * SEQ_PAD)], idx_v)

    tail_col = pl.multiple_of(HEAD + 0 * wid, TILE)
    tail_row = pl.multiple_of(RHEAD + 0 * wid, RTILE)

    def gather_start(gbuf, gsem, c):
        pltpu.make_async_copy(
            table_hbm.at[idx_v.at[pl.ds(c * SEQ_PAD, GROWS)]], gbuf, gsem
        ).start()

    def store_copies(gbuf, ssem, c):
        r0 = pl.multiple_of((base_b + c) * GROWS, RTILE)
        dst = out_hbm.at[pl.ds(r0, GROWS)]
        return (
            pltpu.make_async_copy(
                gbuf.at[:, pl.ds(0, HEAD)],
                dst.at[:, pl.ds(0, HEAD)], ssem),
            pltpu.make_async_copy(
                gbuf.at[:, pl.ds(HEAD, TILE)],
                dst.at[:, pl.ds(tail_col, TILE)], ssem),
        )

    bufs = ((gbuf0, gsem0, ssem0), (gbuf1, gsem1, ssem1))

    for b in range(2):
        gbuf, gsem, _ = bufs[b]
        gather_start(gbuf, gsem, b)

    @pl.loop(0, BATCH_PER_W // 2)
    def _(p):
        for b in range(2):
            gbuf, gsem, ssem = bufs[b]
            c = p * 2 + b

            pltpu.make_async_copy(
                table_hbm.at[idx_v.at[pl.ds(c * SEQ_PAD, GROWS)]], gbuf, gsem
            ).wait()

            for cp in store_copies(gbuf, ssem, c):
                cp.start()

            @pl.when(c + 2 < BATCH_PER_W)
            def _():
                for cp in store_copies(gbuf, ssem, c):
                    cp.wait()
                gather_start(gbuf, gsem, c + 2)

    for b in range(2):
        gbuf, _, ssem = bufs[b]
        for cp in store_copies(gbuf, ssem, BATCH_PER_W - 2 + b):
            cp.wait()


@jax.jit
def _gather(table_pad, idx_pad_flat):
    mesh = plsc.VectorSubcoreMesh(core_axis_name="c", subcore_axis_name="s")
    k = pl.kernel(
        _gather_kernel,
        out_type=jax.ShapeDtypeStruct((BATCH * GROWS, VOCAB), jnp.float32),
        mesh=mesh,
        compiler_params=pltpu.CompilerParams(disable_bounds_checks=True),
        scratch_types=[
            pltpu.VMEM((BATCH_PER_W * SEQ_PAD,), jnp.int32),
            pltpu.VMEM((GROWS, VOCAB_PAD), jnp.float32),
            pltpu.VMEM((GROWS, VOCAB_PAD), jnp.float32),
            pltpu.SemaphoreType.DMA,
            pltpu.SemaphoreType.DMA,
            pltpu.SemaphoreType.DMA,
            pltpu.SemaphoreType.DMA,
        ],
    )
    return k(table_pad, idx_pad_flat)


def kernel(table, idx):
    table_pad = jnp.pad(table, ((0, 0), (0, VOCAB_PAD - VOCAB)))
    idx_pad = jnp.pad(idx, ((0, 0), (0, SEQ_PAD - SEQ)))
    out = _gather(table_pad, idx_pad.reshape(-1))
    return out.reshape(BATCH, GROWS, VOCAB)[:, :SEQ, :]


# speedup vs baseline: 2.4701x; 2.2549x over previous
import jax
import jax.numpy as jnp
from jax import lax
from jax.experimental import pallas as pl
from jax.experimental.pallas import tpu as pltpu
from jax.experimental.pallas import tpu_sc as plsc

VOCAB = 1000
VOCAB_PAD = 1024
BATCH = 1024
SEQ = 50

NC = 2
NS = 16
NW = NC * NS

BATCH_PER_W = BATCH // NW
SEQ_PAD = 64
GROWS = 56

HEAD = 896
TILE = 128
RHEAD = 48
RTILE = 8


def _gather_kernel(
    table_hbm, idx_hbm, out_hbm, idx_v, gbuf0, gbuf1, gsem0, gsem1, ssem0, ssem1
):
    cid = lax.axis_index("c")
    sid = lax.axis_index("s")
    wid = sid * NC + cid
    base_b = wid * BATCH_PER_W

    pltpu.sync_copy(idx_hbm.at[pl.ds(wid * BATCH_PER_W * SEQ_PAD,
                                     BATCH_PER_W * SEQ_PAD)], idx_v)

    tail_col = pl.multiple_of(HEAD + 0 * wid, TILE)
    tail_row = pl.multiple_of(RHEAD + 0 * wid, RTILE)

    def gather_start(gbuf, gsem, c):
        pltpu.make_async_copy(
            table_hbm.at[idx_v.at[pl.ds(c * SEQ_PAD, GROWS)]], gbuf, gsem
        ).start()

    def store_copies(gbuf, ssem, c):
        r0 = pl.multiple_of((base_b + c) * GROWS, RTILE)
        dst = out_hbm.at[pl.ds(r0, GROWS)]
        return (
            pltpu.make_async_copy(
                gbuf.at[:, pl.ds(0, HEAD)],
                dst.at[:, pl.ds(0, HEAD)], ssem),
            pltpu.make_async_copy(
                gbuf.at[:, pl.ds(HEAD, TILE)],
                dst.at[:, pl.ds(tail_col, TILE)], ssem),
        )

    bufs = ((gbuf0, gsem0, ssem0), (gbuf1, gsem1, ssem1))

    for b in range(2):
        gbuf, gsem, _ = bufs[b]
        gather_start(gbuf, gsem, b)

    @pl.loop(0, BATCH_PER_W // 2)
    def _(p):
        for b in range(2):
            gbuf, gsem, ssem = bufs[b]
            c = p * 2 + b

            pltpu.make_async_copy(
                table_hbm.at[idx_v.at[pl.ds(c * SEQ_PAD, GROWS)]], gbuf, gsem
            ).wait()

            for cp in store_copies(gbuf, ssem, c):
                cp.start()

            @pl.when(c + 2 < BATCH_PER_W)
            def _():
                for cp in store_copies(gbuf, ssem, c):
                    cp.wait()
                gather_start(gbuf, gsem, c + 2)

    for b in range(2):
        gbuf, _, ssem = bufs[b]
        for cp in store_copies(gbuf, ssem, BATCH_PER_W - 2 + b):
            cp.wait()


@jax.jit
def _gather(table_pad, idx_pad_flat):
    mesh = plsc.VectorSubcoreMesh(core_axis_name="c", subcore_axis_name="s")
    k = pl.kernel(
        _gather_kernel,
        out_type=jax.ShapeDtypeStruct((BATCH * GROWS, VOCAB), jnp.float32),
        mesh=mesh,
        compiler_params=pltpu.CompilerParams(disable_bounds_checks=True),
        scratch_types=[
            pltpu.VMEM((BATCH_PER_W * SEQ_PAD,), jnp.int32),
            pltpu.VMEM((GROWS, VOCAB_PAD), jnp.float32),
            pltpu.VMEM((GROWS, VOCAB_PAD), jnp.float32),
            pltpu.SemaphoreType.DMA,
            pltpu.SemaphoreType.DMA,
            pltpu.SemaphoreType.DMA,
            pltpu.SemaphoreType.DMA,
        ],
    )
    return k(table_pad, idx_pad_flat)


def kernel(table, idx):
    table_pad = jnp.pad(table, ((0, 0), (0, VOCAB_PAD - VOCAB)))
    fill = (jnp.arange(BATCH, dtype=jnp.int32)[:, None]
            + jnp.arange(SEQ_PAD - SEQ, dtype=jnp.int32)[None, :]) % VOCAB
    idx_pad = jnp.concatenate([idx, fill], axis=1)
    out = _gather(table_pad, idx_pad.reshape(-1))
    return out.reshape(BATCH, GROWS, VOCAB)[:, :SEQ, :]


# speedup vs baseline: 2.5339x; 1.0258x over previous
import jax
import jax.numpy as jnp
from jax import lax
from jax.experimental import pallas as pl
from jax.experimental import layout as jlayout
from jax.experimental.pallas import tpu as pltpu
from jax.experimental.pallas import tpu_sc as plsc

VOCAB = 1000
VOCAB_PAD = 1024
BATCH = 1024
SEQ = 50

NC = 2
NS = 16
NW = NC * NS

BATCH_PER_W = BATCH // NW
SEQ_PAD = 64
GROWS = 50

HEAD = 896
TILE = 128
RTILE = 8


def _gather_kernel(
    table_hbm, idx_hbm, out_hbm, idx_v, gbuf0, gbuf1, gsem0, gsem1, ssem0, ssem1
):
    cid = lax.axis_index("c")
    sid = lax.axis_index("s")
    wid = sid * NC + cid
    base_b = wid * BATCH_PER_W

    pltpu.sync_copy(idx_hbm.at[pl.ds(wid * BATCH_PER_W * SEQ_PAD,
                                     BATCH_PER_W * SEQ_PAD)], idx_v)

    tail_col = pl.multiple_of(HEAD + 0 * wid, TILE)

    def gather_start(gbuf, gsem, c):
        pltpu.make_async_copy(
            table_hbm.at[idx_v.at[pl.ds(c * SEQ_PAD, GROWS)]], gbuf, gsem
        ).start()

    def store_copies(gbuf, ssem, c):
        dst = out_hbm.at[base_b + c]
        return (
            pltpu.make_async_copy(
                gbuf.at[:, pl.ds(0, HEAD)],
                dst.at[:, pl.ds(0, HEAD)], ssem),
            pltpu.make_async_copy(
                gbuf.at[:, pl.ds(HEAD, TILE)],
                dst.at[:, pl.ds(tail_col, TILE)], ssem),
        )

    bufs = ((gbuf0, gsem0, ssem0), (gbuf1, gsem1, ssem1))

    for b in range(2):
        gbuf, gsem, _ = bufs[b]
        gather_start(gbuf, gsem, b)

    @pl.loop(0, BATCH_PER_W // 2)
    def _(p):
        for b in range(2):
            gbuf, gsem, ssem = bufs[b]
            c = p * 2 + b

            pltpu.make_async_copy(
                table_hbm.at[idx_v.at[pl.ds(c * SEQ_PAD, GROWS)]], gbuf, gsem
            ).wait()

            for cp in store_copies(gbuf, ssem, c):
                cp.start()

            @pl.when(c + 2 < BATCH_PER_W)
            def _():
                for cp in store_copies(gbuf, ssem, c):
                    cp.wait()
                gather_start(gbuf, gsem, c + 2)

    for b in range(2):
        gbuf, _, ssem = bufs[b]
        for cp in store_copies(gbuf, ssem, BATCH_PER_W - 2 + b):
            cp.wait()


def _gather(table_pad, idx_pad_flat):
    mesh = plsc.VectorSubcoreMesh(core_axis_name="c", subcore_axis_name="s")
    k = pl.kernel(
        _gather_kernel,
        out_type=jax.ShapeDtypeStruct((BATCH, SEQ, VOCAB), jnp.float32),
        mesh=mesh,
        compiler_params=pltpu.CompilerParams(disable_bounds_checks=True),
        scratch_types=[
            pltpu.VMEM((BATCH_PER_W * SEQ_PAD,), jnp.int32),
            pltpu.VMEM((GROWS, VOCAB_PAD), jnp.float32),
            pltpu.VMEM((GROWS, VOCAB_PAD), jnp.float32),
            pltpu.SemaphoreType.DMA,
            pltpu.SemaphoreType.DMA,
            pltpu.SemaphoreType.DMA,
            pltpu.SemaphoreType.DMA,
        ],
    )
    return k(table_pad, idx_pad_flat)


def _kernel_impl(table, idx):
    table_pad = jnp.pad(table, ((0, 0), (0, VOCAB_PAD - VOCAB)))
    fill = (jnp.arange(BATCH, dtype=jnp.int32)[:, None]
            + jnp.arange(SEQ_PAD - SEQ, dtype=jnp.int32)[None, :]) % VOCAB
    idx_pad = jnp.concatenate([idx, fill], axis=1)
    out = _gather(table_pad, idx_pad.reshape(-1))
    return lax.optimization_barrier(out)


_jitted = None


def kernel(table, idx):
    global _jitted
    if _jitted is None:
        fmt = jlayout.Format(
            jlayout.Layout(major_to_minor=(0, 1, 2)),
            jax.sharding.SingleDeviceSharding(jax.devices()[0]),
        )
        _jitted = jax.jit(_kernel_impl, out_shardings=fmt)
    return _jitted(table, idx)
